# baseline (speedup 1.0000x reference)
import jax
import jax.numpy as jnp
from jax.experimental import pallas as pl
from jax.experimental.pallas import tpu as pltpu


def kernel(x):
    B, C, H, W = x.shape
    R = B * C
    CH = 16
    NS = 16
    P = 64 // CH
    NCH = R // CH
    xf = x.reshape(R, H, W)

    def body(x_hbm, o_hbm, buf, lsem, ssem):
        for s in range(0, NS, P):
            buf[pl.ds(CH * s + 1, 1)] = jnp.full((1, H, W), 4.0, x.dtype)

        def loads(c):
            s = c % NS
            b0, r0 = CH * s, CH * c
            if c % P == 0:
                return [
                    pltpu.make_async_copy(
                        x_hbm.at[pl.ds(r0, 1)], buf.at[pl.ds(b0, 1)],
                        lsem.at[s]),
                    pltpu.make_async_copy(
                        x_hbm.at[pl.ds(r0 + 2, CH - 2)],
                        buf.at[pl.ds(b0 + 2, CH - 2)], lsem.at[s]),
                ]
            return [pltpu.make_async_copy(
                x_hbm.at[pl.ds(r0, CH)], buf.at[pl.ds(b0, CH)], lsem.at[s])]

        def store(c):
            s = c % NS
            return pltpu.make_async_copy(
                buf.at[pl.ds(CH * s, CH)], o_hbm.at[pl.ds(CH * c, CH)],
                ssem.at[s])

        pending = {}
        for c in range(NS):
            pending[c] = loads(c)
            for d in pending[c]:
                d.start()
        stores = {}
        for c in range(NCH):
            for d in pending.pop(c):
                d.wait()
            stores[c] = store(c)
            stores[c].start()
            if c + NS < NCH:
                stores[c].wait()
                pending[c + NS] = loads(c + NS)
                for d in pending[c + NS]:
                    d.start()
        for c in range(NCH - NS, NCH):
            stores[c].wait()

    out = pl.pallas_call(
        body,
        in_specs=[pl.BlockSpec(memory_space=pl.ANY)],
        out_specs=pl.BlockSpec(memory_space=pl.ANY),
        out_shape=jax.ShapeDtypeStruct((R, H, W), x.dtype),
        scratch_shapes=[
            pltpu.VMEM((NS * CH, H, W), x.dtype),
            pltpu.SemaphoreType.DMA((NS,)),
            pltpu.SemaphoreType.DMA((NS,)),
        ],
        compiler_params=pltpu.CompilerParams(
            vmem_limit_bytes=100 * 1024 * 1024),
    )(xf)
    return out.reshape(B, C, H, W)


# speedup vs baseline: 1.0073x; 1.0073x over previous
import jax
import jax.numpy as jnp
from jax.experimental import pallas as pl
from jax.experimental.pallas import tpu as pltpu


def kernel(x):
    B, C, H, W = x.shape
    R = B * C
    CH = 64
    NS = 4
    P = 64 // CH
    NCH = R // CH
    xf = x.reshape(R, H, W)

    def body(x_hbm, o_hbm, buf, lsem, ssem):
        for s in range(0, NS, P):
            buf[pl.ds(CH * s + 1, 1)] = jnp.full((1, H, W), 4.0, x.dtype)

        def loads(c):
            s = c % NS
            b0, r0 = CH * s, CH * c
            if c % P == 0:
                return [
                    pltpu.make_async_copy(
                        x_hbm.at[pl.ds(r0, 1)], buf.at[pl.ds(b0, 1)],
                        lsem.at[s]),
                    pltpu.make_async_copy(
                        x_hbm.at[pl.ds(r0 + 2, CH - 2)],
                        buf.at[pl.ds(b0 + 2, CH - 2)], lsem.at[s]),
                ]
            return [pltpu.make_async_copy(
                x_hbm.at[pl.ds(r0, CH)], buf.at[pl.ds(b0, CH)], lsem.at[s])]

        def store(c):
            s = c % NS
            return pltpu.make_async_copy(
                buf.at[pl.ds(CH * s, CH)], o_hbm.at[pl.ds(CH * c, CH)],
                ssem.at[s])

        pending = {}
        for c in range(NS):
            pending[c] = loads(c)
            for d in pending[c]:
                d.start()
        stores = {}
        for c in range(NCH):
            for d in pending.pop(c):
                d.wait()
            stores[c] = store(c)
            stores[c].start()
            if c + NS < NCH:
                stores[c].wait()
                pending[c + NS] = loads(c + NS)
                for d in pending[c + NS]:
                    d.start()
        for c in range(NCH - NS, NCH):
            stores[c].wait()

    out = pl.pallas_call(
        body,
        in_specs=[pl.BlockSpec(memory_space=pl.ANY)],
        out_specs=pl.BlockSpec(memory_space=pl.ANY),
        out_shape=jax.ShapeDtypeStruct((R, H, W), x.dtype),
        scratch_shapes=[
            pltpu.VMEM((NS * CH, H, W), x.dtype),
            pltpu.SemaphoreType.DMA((NS,)),
            pltpu.SemaphoreType.DMA((NS,)),
        ],
        compiler_params=pltpu.CompilerParams(
            vmem_limit_bytes=100 * 1024 * 1024),
    )(xf)
    return out.reshape(B, C, H, W)


# speedup vs baseline: 1.0084x; 1.0011x over previous
import jax
import jax.numpy as jnp
from jax.experimental import pallas as pl
from jax.experimental.pallas import tpu as pltpu


def kernel(x):
    B, C, H, W = x.shape
    R = B * C
    CH = 32
    NS = 8
    HC = CH // 2
    NCH = R // CH
    xf = x.reshape(R, H, W)

    def body(x_hbm, o_hbm, buf, lsem, ssem):
        for s in range(0, NS, 2):
            buf[pl.ds(CH * s + 1, 1)] = jnp.full((1, H, W), 4.0, x.dtype)

        def loads(c):
            s = c % NS
            b0, r0 = CH * s, CH * c
            first = ([pltpu.make_async_copy(
                          x_hbm.at[pl.ds(r0, 1)], buf.at[pl.ds(b0, 1)],
                          lsem.at[s]),
                      pltpu.make_async_copy(
                          x_hbm.at[pl.ds(r0 + 2, HC - 2)],
                          buf.at[pl.ds(b0 + 2, HC - 2)], lsem.at[s])]
                     if c % 2 == 0 else
                     [pltpu.make_async_copy(
                         x_hbm.at[pl.ds(r0, HC)], buf.at[pl.ds(b0, HC)],
                         lsem.at[s])])
            return first + [pltpu.make_async_copy(
                x_hbm.at[pl.ds(r0 + HC, HC)], buf.at[pl.ds(b0 + HC, HC)],
                lsem.at[s])]

        def stores(c):
            s = c % NS
            return [pltpu.make_async_copy(
                        buf.at[pl.ds(CH * s + h, HC)],
                        o_hbm.at[pl.ds(CH * c + h, HC)], ssem.at[s])
                    for h in (0, HC)]

        pending = {}
        for c in range(NS):
            pending[c] = loads(c)
            for d in pending[c]:
                d.start()
        outs = {}
        for c in range(NCH):
            for d in pending.pop(c):
                d.wait()
            outs[c] = stores(c)
            for d in outs[c]:
                d.start()
            if c + NS < NCH:
                for d in outs[c]:
                    d.wait()
                pending[c + NS] = loads(c + NS)
                for d in pending[c + NS]:
                    d.start()
        for c in range(NCH - NS, NCH):
            for d in outs[c]:
                d.wait()

    out = pl.pallas_call(
        body,
        in_specs=[pl.BlockSpec(memory_space=pl.ANY)],
        out_specs=pl.BlockSpec(memory_space=pl.ANY),
        out_shape=jax.ShapeDtypeStruct((R, H, W), x.dtype),
        scratch_shapes=[
            pltpu.VMEM((NS * CH, H, W), x.dtype),
            pltpu.SemaphoreType.DMA((NS,)),
            pltpu.SemaphoreType.DMA((NS,)),
        ],
        compiler_params=pltpu.CompilerParams(
            vmem_limit_bytes=100 * 1024 * 1024),
    )(xf)
    return out.reshape(B, C, H, W)


# speedup vs baseline: 1.0088x; 1.0003x over previous
import jax
import jax.numpy as jnp
from jax.experimental import pallas as pl
from jax.experimental.pallas import tpu as pltpu


def kernel(x):
    B, C, H, W = x.shape
    R = B * C
    CH = 32
    NS = 8
    NCH = R // CH
    xf = x.reshape(R, H, W)

    def body(x_hbm, o_hbm, buf, lsem, ssem):
        def loads(c):
            s = c % NS
            b0, r0 = CH * s, CH * c
            if c % 2 == 0:
                return [
                    pltpu.make_async_copy(
                        x_hbm.at[pl.ds(r0, 1)], buf.at[pl.ds(b0, 1)],
                        lsem.at[s]),
                    pltpu.make_async_copy(
                        x_hbm.at[pl.ds(r0 + 2, CH - 2)],
                        buf.at[pl.ds(b0 + 2, CH - 2)], lsem.at[s]),
                ]
            return [pltpu.make_async_copy(
                x_hbm.at[pl.ds(r0, CH)], buf.at[pl.ds(b0, CH)], lsem.at[s])]

        def store(c):
            s = c % NS
            return pltpu.make_async_copy(
                buf.at[pl.ds(CH * s, CH)], o_hbm.at[pl.ds(CH * c, CH)],
                ssem.at[s])

        pending = {}
        for c in range(NS):
            pending[c] = loads(c)
            for d in pending[c]:
                d.start()
        for s in range(0, NS, 2):
            buf[pl.ds(CH * s + 1, 1)] = jnp.full((1, H, W), 4.0, x.dtype)
        stores = {}
        for c in range(NCH):
            for d in pending.pop(c):
                d.wait()
            stores[c] = store(c)
            stores[c].start()
            if c + NS < NCH:
                stores[c].wait()
                pending[c + NS] = loads(c + NS)
                for d in pending[c + NS]:
                    d.start()
        for c in range(NCH - NS, NCH):
            stores[c].wait()

    out = pl.pallas_call(
        body,
        in_specs=[pl.BlockSpec(memory_space=pl.ANY)],
        out_specs=pl.BlockSpec(memory_space=pl.ANY),
        out_shape=jax.ShapeDtypeStruct((R, H, W), x.dtype),
        scratch_shapes=[
            pltpu.VMEM((NS * CH, H, W), x.dtype),
            pltpu.SemaphoreType.DMA((NS,)),
            pltpu.SemaphoreType.DMA((NS,)),
        ],
        compiler_params=pltpu.CompilerParams(
            vmem_limit_bytes=100 * 1024 * 1024),
    )(xf)
    return out.reshape(B, C, H, W)
